# baseline (speedup 1.0000x reference)
import functools
import math

import jax
import jax.numpy as jnp
from jax.experimental import pallas as pl
from jax.experimental.pallas import tpu as pltpu

DEPTH = 3
RATIO = 0.5


def _mm_body(a_ref, b_ref, o_ref, acc_ref, *, nk):
    @pl.when(pl.program_id(2) == 0)
    def _():
        acc_ref[...] = jnp.zeros_like(acc_ref)

    a = a_ref[...]
    b = b_ref[...]
    acc_ref[...] += jnp.dot(a.astype(jnp.float32), b.astype(jnp.float32),
                            preferred_element_type=jnp.float32)

    @pl.when(pl.program_id(2) == nk - 1)
    def _():
        o_ref[...] = acc_ref[...]


def _mm(a, b, bm=512, bn=512, bk=512):
    m, k = a.shape
    k2, n = b.shape
    bm = min(bm, m)
    bn = min(bn, n)
    bk = min(bk, k)
    grid = (m // bm, n // bn, k // bk)
    return pl.pallas_call(
        functools.partial(_mm_body, nk=grid[2]),
        out_shape=jax.ShapeDtypeStruct((m, n), jnp.float32),
        grid=grid,
        in_specs=[
            pl.BlockSpec((bm, bk), lambda i, j, h: (i, h)),
            pl.BlockSpec((bk, bn), lambda i, j, h: (h, j)),
        ],
        out_specs=pl.BlockSpec((bm, bn), lambda i, j, h: (i, j)),
        scratch_shapes=[pltpu.VMEM((bm, bn), jnp.float32)],
    )(a, b)


def _aug_body(a_ref, b_ref, o_ref, acc_ref, *, nk, bm, bn, bk):
    i = pl.program_id(0)
    j = pl.program_id(1)
    h = pl.program_id(2)

    @pl.when(h == 0)
    def _():
        acc_ref[...] = jnp.zeros_like(acc_ref)

    acc_ref[...] += jnp.dot(a_ref[...], b_ref[...],
                            preferred_element_type=jnp.float32)

    @pl.when(h == nk - 1)
    def _():
        acc = acc_ref[...]

        @pl.when(i == j)
        def _():
            r = jax.lax.broadcasted_iota(jnp.int32, (bm, bn), 0)
            c = jax.lax.broadcasted_iota(jnp.int32, (bm, bn), 1)
            acc_ref[...] = jnp.where(i * bm + r == j * bn + c, 0.0, acc)

        o_ref[...] = acc_ref[...].astype(jnp.bfloat16)


def _augment(a_bf):
    n = a_bf.shape[0]
    bm = bn = min(2048, n)
    bk = min(512, n)
    grid = (n // bm, n // bn, n // bk)
    return pl.pallas_call(
        functools.partial(_aug_body, nk=grid[2], bm=bm, bn=bn, bk=bk),
        out_shape=jax.ShapeDtypeStruct((n, n), jnp.bfloat16),
        grid=grid,
        in_specs=[
            pl.BlockSpec((bm, bk), lambda i, j, h: (i, h)),
            pl.BlockSpec((bk, bn), lambda i, j, h: (h, j)),
        ],
        out_specs=pl.BlockSpec((bm, bn), lambda i, j, h: (i, j)),
        scratch_shapes=[pltpu.VMEM((bm, bn), jnp.float32)],
    )(a_bf, a_bf)


def _stats_body(a_ref, r_ref, c_ref, abf_ref, *, blk):
    i = pl.program_id(0)
    k = pl.program_id(1)
    a = a_ref[...].astype(jnp.float32)

    @pl.when(i == k)
    def _():
        rr = jax.lax.broadcasted_iota(jnp.int32, (blk, blk), 0)
        cc = jax.lax.broadcasted_iota(jnp.int32, (blk, blk), 1)
        abf_ref[...] = jnp.where(rr == cc, 1.0, a).astype(jnp.bfloat16)

    @pl.when(i != k)
    def _():
        abf_ref[...] = a.astype(jnp.bfloat16)

    @pl.when(k == 0)
    def _():
        r_ref[...] = jnp.zeros_like(r_ref)
        c_ref[...] = jnp.zeros_like(c_ref)

    r_ref[...] += jnp.sum(a, axis=1, keepdims=True) + jnp.zeros(
        (blk, 128), jnp.float32)

    @pl.when(i == k)
    def _():
        rr = jax.lax.broadcasted_iota(jnp.int32, (blk, blk), 0)
        cc = jax.lax.broadcasted_iota(jnp.int32, (blk, blk), 1)
        c_ref[...] += jnp.sum(jnp.where(rr == cc, a, 0.0), axis=1,
                              keepdims=True) + jnp.zeros((blk, 128),
                                                         jnp.float32)


def _stats(a):
    n = a.shape[0]
    blk = min(512, n)
    r, c, abf = pl.pallas_call(
        functools.partial(_stats_body, blk=blk),
        out_shape=[
            jax.ShapeDtypeStruct((n, 128), jnp.float32),
            jax.ShapeDtypeStruct((n, 128), jnp.float32),
            jax.ShapeDtypeStruct((n, n), jnp.bfloat16),
        ],
        grid=(n // blk, n // blk),
        in_specs=[pl.BlockSpec((blk, blk), lambda i, k: (i, k))],
        out_specs=[
            pl.BlockSpec((blk, 128), lambda i, k: (i, 0)),
            pl.BlockSpec((blk, 128), lambda i, k: (i, 0)),
            pl.BlockSpec((blk, blk), lambda i, k: (i, k)),
        ],
    )(a)
    return r[:, 0], c[:, 0], abf


def _norm_vecs(r, c):
    extra = jnp.where(c == 0, 2.0, 0.0)
    deg = r + extra
    dinv = jnp.where(deg > 0, jax.lax.rsqrt(deg), 0.0)
    coeff = (c - 1.0 + extra) * dinv * dinv
    return dinv, coeff


def _gcn_conv(a_raw, dinv, coeff, x, W, b, relu, row_scale=None):
    if row_scale is not None:
        x = x * row_scale[:, None]
    z = _mm(x, W, bn=128)
    zs = z * dinv[:, None]
    y = _mm(a_raw, zs, bn=128) * dinv[:, None] + coeff[:, None] * z + b
    if relu:
        y = jax.nn.relu(y)
    return y


def kernel(x, edge_index, W_down0, b_down0, W_down1, b_down1, W_down2,
           b_down2, W_down3, b_down3, p_pool1, p_pool2, p_pool3,
           W_up0, b_up0, W_up1, b_up1, W_up2, b_up2):
    n = x.shape[0]
    A32 = jnp.zeros((n, n), jnp.float32).at[edge_index[1], edge_index[0]].add(
        jnp.ones((edge_index.shape[1],), jnp.float32))

    r, c, A = _stats(A32)
    dinv, coeff = _norm_vecs(r, c)
    x = _gcn_conv(A, dinv, coeff, x, W_down0, b_down0, relu=True)

    xs = [x]
    As = [A]
    norms = [(dinv, coeff)]
    perms = []
    Wd = [(W_down1, b_down1), (W_down2, b_down2), (W_down3, b_down3)]
    ps = [p_pool1, p_pool2, p_pool3]

    for i in range(DEPTH):
        A2 = _augment(A)
        p = ps[i]
        pn = p / jnp.linalg.norm(p)
        P = jnp.zeros((128, 128), jnp.float32).at[:, 0].set(pn)
        score = _mm(x, P, bn=128)[:, 0]
        k = int(math.ceil(RATIO * x.shape[0]))
        vals, perm = jax.lax.top_k(score, k)
        scale = jnp.tanh(vals)
        A = A2[perm][:, perm]

        r, c, A = _stats(A)
        dinv, coeff = _norm_vecs(r, c)
        xg = x[perm]
        x = _gcn_conv(A, dinv, coeff, xg, Wd[i][0], Wd[i][1], relu=True,
                      row_scale=scale)
        if i < DEPTH - 1:
            xs.append(x)
            As.append(A)
            norms.append((dinv, coeff))
        perms.append(perm)

    Wu = [(W_up0, b_up0), (W_up1, b_up1), (W_up2, b_up2)]
    for i in range(DEPTH):
        j = DEPTH - 1 - i
        res = xs[j]
        A = As[j]
        dinv, coeff = norms[j]
        perm = perms[j]
        Wt, bt = Wu[i]
        h = _mm(res, Wt[:128], bn=128) + jnp.zeros(
            (res.shape[0], Wt.shape[1]), jnp.float32).at[perm].set(
                _mm(x, Wt[128:], bn=128))
        hs = h * dinv[:, None]
        y = _mm(A, hs, bn=128) * dinv[:, None] + coeff[:, None] * h + bt
        if i < DEPTH - 1:
            y = jax.nn.relu(y)
        x = y
    return x


# speedup vs baseline: 1.0229x; 1.0229x over previous
import functools
import math

import jax
import jax.numpy as jnp
from jax.experimental import pallas as pl
from jax.experimental.pallas import tpu as pltpu

DEPTH = 3
RATIO = 0.5


def _mm_body(a_ref, b_ref, o_ref, acc_ref, *, nk):
    @pl.when(pl.program_id(2) == 0)
    def _():
        acc_ref[...] = jnp.zeros_like(acc_ref)

    a = a_ref[...]
    b = b_ref[...]
    acc_ref[...] += jnp.dot(a.astype(jnp.float32), b.astype(jnp.float32),
                            preferred_element_type=jnp.float32)

    @pl.when(pl.program_id(2) == nk - 1)
    def _():
        o_ref[...] = acc_ref[...]


def _mm(a, b, bm=512, bn=512, bk=512):
    m, k = a.shape
    k2, n = b.shape
    bm = min(bm, m)
    bn = min(bn, n)
    bk = min(bk, k)
    grid = (m // bm, n // bn, k // bk)
    return pl.pallas_call(
        functools.partial(_mm_body, nk=grid[2]),
        out_shape=jax.ShapeDtypeStruct((m, n), jnp.float32),
        grid=grid,
        in_specs=[
            pl.BlockSpec((bm, bk), lambda i, j, h: (i, h)),
            pl.BlockSpec((bk, bn), lambda i, j, h: (h, j)),
        ],
        out_specs=pl.BlockSpec((bm, bn), lambda i, j, h: (i, j)),
        scratch_shapes=[pltpu.VMEM((bm, bn), jnp.float32)],
    )(a, b)


def _aug_body(a_ref, b_ref, o_ref, *, bm, bn):
    i = pl.program_id(0)
    j = pl.program_id(1)
    acc = jnp.dot(a_ref[...], b_ref[...], preferred_element_type=jnp.float32)

    r = jax.lax.broadcasted_iota(jnp.int32, (bm, bn), 0)
    c = jax.lax.broadcasted_iota(jnp.int32, (bm, bn), 1)
    acc = jnp.where(i * bm + r == j * bn + c, 0.0, acc)
    o_ref[...] = acc.astype(jnp.bfloat16)


def _augment(a_bf):
    n = a_bf.shape[0]
    bm = bn = min(1024, n)
    grid = (n // bm, n // bn)
    return pl.pallas_call(
        functools.partial(_aug_body, bm=bm, bn=bn),
        out_shape=jax.ShapeDtypeStruct((n, n), jnp.bfloat16),
        grid=grid,
        in_specs=[
            pl.BlockSpec((bm, n), lambda i, j: (i, 0)),
            pl.BlockSpec((n, bn), lambda i, j: (0, j)),
        ],
        out_specs=pl.BlockSpec((bm, bn), lambda i, j: (i, j)),
    )(a_bf, a_bf)


def _stats_body(a_ref, r_ref, c_ref, abf_ref, *, blk):
    i = pl.program_id(0)
    k = pl.program_id(1)
    a = a_ref[...].astype(jnp.float32)

    @pl.when(i == k)
    def _():
        rr = jax.lax.broadcasted_iota(jnp.int32, (blk, blk), 0)
        cc = jax.lax.broadcasted_iota(jnp.int32, (blk, blk), 1)
        abf_ref[...] = jnp.where(rr == cc, 1.0, a).astype(jnp.bfloat16)

    @pl.when(i != k)
    def _():
        abf_ref[...] = a.astype(jnp.bfloat16)

    @pl.when(k == 0)
    def _():
        r_ref[...] = jnp.zeros_like(r_ref)
        c_ref[...] = jnp.zeros_like(c_ref)

    r_ref[...] += jnp.sum(a, axis=1, keepdims=True) + jnp.zeros(
        (blk, 128), jnp.float32)

    @pl.when(i == k)
    def _():
        rr = jax.lax.broadcasted_iota(jnp.int32, (blk, blk), 0)
        cc = jax.lax.broadcasted_iota(jnp.int32, (blk, blk), 1)
        c_ref[...] += jnp.sum(jnp.where(rr == cc, a, 0.0), axis=1,
                              keepdims=True) + jnp.zeros((blk, 128),
                                                         jnp.float32)


def _stats(a):
    n = a.shape[0]
    blk = min(512, n)
    r, c, abf = pl.pallas_call(
        functools.partial(_stats_body, blk=blk),
        out_shape=[
            jax.ShapeDtypeStruct((n, 128), jnp.float32),
            jax.ShapeDtypeStruct((n, 128), jnp.float32),
            jax.ShapeDtypeStruct((n, n), jnp.bfloat16),
        ],
        grid=(n // blk, n // blk),
        in_specs=[pl.BlockSpec((blk, blk), lambda i, k: (i, k))],
        out_specs=[
            pl.BlockSpec((blk, 128), lambda i, k: (i, 0)),
            pl.BlockSpec((blk, 128), lambda i, k: (i, 0)),
            pl.BlockSpec((blk, blk), lambda i, k: (i, k)),
        ],
    )(a)
    return r[:, 0], c[:, 0], abf


def _norm_vecs(r, c):
    extra = jnp.where(c == 0, 2.0, 0.0)
    deg = r + extra
    dinv = jnp.where(deg > 0, jax.lax.rsqrt(deg), 0.0)
    coeff = (c - 1.0 + extra) * dinv * dinv
    return dinv, coeff


def _gcn_conv(a_raw, dinv, coeff, x, W, b, relu, row_scale=None):
    if row_scale is not None:
        x = x * row_scale[:, None]
    z = _mm(x, W, bn=128)
    zs = z * dinv[:, None]
    y = _mm(a_raw, zs, bn=128) * dinv[:, None] + coeff[:, None] * z + b
    if relu:
        y = jax.nn.relu(y)
    return y


def kernel(x, edge_index, W_down0, b_down0, W_down1, b_down1, W_down2,
           b_down2, W_down3, b_down3, p_pool1, p_pool2, p_pool3,
           W_up0, b_up0, W_up1, b_up1, W_up2, b_up2):
    n = x.shape[0]
    A32 = jnp.zeros((n, n), jnp.float32).at[edge_index[1], edge_index[0]].add(
        jnp.ones((edge_index.shape[1],), jnp.float32))

    r, c, A = _stats(A32)
    dinv, coeff = _norm_vecs(r, c)
    x = _gcn_conv(A, dinv, coeff, x, W_down0, b_down0, relu=True)

    xs = [x]
    As = [A]
    norms = [(dinv, coeff)]
    perms = []
    Wd = [(W_down1, b_down1), (W_down2, b_down2), (W_down3, b_down3)]
    ps = [p_pool1, p_pool2, p_pool3]

    for i in range(DEPTH):
        A2 = _augment(A)
        p = ps[i]
        pn = p / jnp.linalg.norm(p)
        P = jnp.zeros((128, 128), jnp.float32).at[:, 0].set(pn)
        score = _mm(x, P, bn=128)[:, 0]
        k = int(math.ceil(RATIO * x.shape[0]))
        vals, perm = jax.lax.top_k(score, k)
        scale = jnp.tanh(vals)
        A = A2[perm][:, perm]

        r, c, A = _stats(A)
        dinv, coeff = _norm_vecs(r, c)
        xg = x[perm]
        x = _gcn_conv(A, dinv, coeff, xg, Wd[i][0], Wd[i][1], relu=True,
                      row_scale=scale)
        if i < DEPTH - 1:
            xs.append(x)
            As.append(A)
            norms.append((dinv, coeff))
        perms.append(perm)

    Wu = [(W_up0, b_up0), (W_up1, b_up1), (W_up2, b_up2)]
    for i in range(DEPTH):
        j = DEPTH - 1 - i
        res = xs[j]
        A = As[j]
        dinv, coeff = norms[j]
        perm = perms[j]
        Wt, bt = Wu[i]
        h = _mm(res, Wt[:128], bn=128) + jnp.zeros(
            (res.shape[0], Wt.shape[1]), jnp.float32).at[perm].set(
                _mm(x, Wt[128:], bn=128))
        hs = h * dinv[:, None]
        y = _mm(A, hs, bn=128) * dinv[:, None] + coeff[:, None] * h + bt
        if i < DEPTH - 1:
            y = jax.nn.relu(y)
        x = y
    return x


# speedup vs baseline: 1.0980x; 1.0734x over previous
import functools
import math

import jax
import jax.numpy as jnp
from jax import lax
from jax.experimental import pallas as pl
from jax.experimental.pallas import tpu as pltpu
from jax.experimental.pallas import tpu_sc as plsc

DEPTH = 3
RATIO = 0.5


def _mm_body(a_ref, b_ref, o_ref, acc_ref, *, nk):
    @pl.when(pl.program_id(2) == 0)
    def _():
        acc_ref[...] = jnp.zeros_like(acc_ref)

    a = a_ref[...]
    b = b_ref[...]
    acc_ref[...] += jnp.dot(a.astype(jnp.float32), b.astype(jnp.float32),
                            preferred_element_type=jnp.float32)

    @pl.when(pl.program_id(2) == nk - 1)
    def _():
        o_ref[...] = acc_ref[...]


def _mm(a, b, bm=512, bn=512, bk=512):
    m, k = a.shape
    k2, n = b.shape
    bm = min(bm, m)
    bn = min(bn, n)
    bk = min(bk, k)
    grid = (m // bm, n // bn, k // bk)
    return pl.pallas_call(
        functools.partial(_mm_body, nk=grid[2]),
        out_shape=jax.ShapeDtypeStruct((m, n), jnp.float32),
        grid=grid,
        in_specs=[
            pl.BlockSpec((bm, bk), lambda i, j, h: (i, h)),
            pl.BlockSpec((bk, bn), lambda i, j, h: (h, j)),
        ],
        out_specs=pl.BlockSpec((bm, bn), lambda i, j, h: (i, j)),
        scratch_shapes=[pltpu.VMEM((bm, bn), jnp.float32)],
    )(a, b)


def _aug_body(a_ref, b_ref, o_ref, *, bm, bn):
    i = pl.program_id(0)
    j = pl.program_id(1)
    acc = jnp.dot(a_ref[...].astype(jnp.bfloat16),
                  b_ref[...].astype(jnp.bfloat16),
                  preferred_element_type=jnp.float32)

    r = jax.lax.broadcasted_iota(jnp.int32, (bm, bn), 0)
    c = jax.lax.broadcasted_iota(jnp.int32, (bm, bn), 1)
    acc = jnp.where(i * bm + r == j * bn + c, 0.0, acc)
    o_ref[...] = acc


def _augment(a_any):
    n = a_any.shape[0]
    bm = bn = min(1024, n) if a_any.dtype == jnp.bfloat16 else min(512, n)
    grid = (n // bm, n // bn)
    return pl.pallas_call(
        functools.partial(_aug_body, bm=bm, bn=bn),
        out_shape=jax.ShapeDtypeStruct((n, n), jnp.float32),
        grid=grid,
        in_specs=[
            pl.BlockSpec((bm, n), lambda i, j: (i, 0)),
            pl.BlockSpec((n, bn), lambda i, j: (0, j)),
        ],
        out_specs=pl.BlockSpec((bm, bn), lambda i, j: (i, j)),
    )(a_any, a_any)


def _pool_sc(A2, x, perm):
    n = A2.shape[0]
    k = perm.shape[0]
    NW = 32
    rows_pw = k // NW
    nch = rows_pw // 16
    k16 = k // 16
    mesh = plsc.VectorSubcoreMesh(core_axis_name="c", subcore_axis_name="s")

    @functools.partial(
        pl.kernel,
        mesh=mesh,
        compiler_params=pltpu.CompilerParams(needs_layout_passes=False),
        out_type=[
            jax.ShapeDtypeStruct((k, k), jnp.float32),
            jax.ShapeDtypeStruct((k, 128), jnp.float32),
            jax.ShapeDtypeStruct((k,), jnp.float32),
        ],
        scratch_types=[
            pltpu.VMEM((k,), jnp.int32),
            pltpu.VMEM((16,), jnp.int32),
            pltpu.VMEM((rows_pw,), jnp.int32),
            pltpu.VMEM((16, n), jnp.float32),
            pltpu.VMEM((k,), jnp.float32),
            pltpu.VMEM((k,), jnp.float32),
            pltpu.VMEM((rows_pw,), jnp.float32),
            pltpu.VMEM((rows_pw, 128), jnp.float32),
            pltpu.SemaphoreType.DMA,
            pltpu.SemaphoreType.DMA,
            pltpu.SemaphoreType.DMA,
        ],
    )
    def body(a2_h, x_h, perm_h, ap_o, xg_o, rs_o,
             colidx, idxc, xidx, rowbuf, orow0, orow1, rsv, xgbuf,
             sg, s0, s1):
        wid = lax.axis_index("s") * 2 + lax.axis_index("c")
        base = wid * rows_pw
        pltpu.sync_copy(perm_h, colidx)
        pltpu.sync_copy(perm_h.at[pl.ds(base, rows_pw)], xidx)
        pltpu.async_copy(x_h.at[xidx], xgbuf, sg).wait()
        pltpu.sync_copy(xgbuf, xg_o.at[pl.ds(base, rows_pw)])

        iot = lax.iota(jnp.int32, 16)
        obufs = [orow0, orow1]
        sems = [s0, s1]
        pend = [None, None]
        for ch in range(nch):
            cbase = base + ch * 16
            pltpu.sync_copy(perm_h.at[pl.ds(cbase, 16)], idxc)
            pltpu.async_copy(a2_h.at[idxc], rowbuf, sg).wait()
            gacc = jnp.zeros((16,), jnp.float32)
            for r in range(16):
                ob = obufs[r % 2]
                if pend[r % 2] is not None:
                    pend[r % 2].wait()
                rvec = jnp.full((16,), r, jnp.int32)

                def jbody(jj, acc, ob=ob, rvec=rvec):
                    cidx = colidx[pl.ds(jj * 16, 16)]
                    g = plsc.load_gather(rowbuf, [rvec, cidx])
                    ob[pl.ds(jj * 16, 16)] = g
                    return acc + g

                acc = lax.fori_loop(0, k16, jbody,
                                    jnp.zeros((16,), jnp.float32))
                R = cbase + r
                plsc.store_scatter(ob, [jnp.full((16,), R, jnp.int32)],
                                   jnp.full((16,), 1.0, jnp.float32),
                                   mask=iot == 0)
                gacc = gacc + jnp.where(iot == r, jnp.sum(acc), 0.0)
                pend[r % 2] = pltpu.async_copy(ob, ap_o.at[R], sems[r % 2])
            rsv[pl.ds(ch * 16, 16)] = gacc
        for p in pend:
            if p is not None:
                p.wait()
        pltpu.sync_copy(rsv, rs_o.at[pl.ds(base, rows_pw)])

    return body(A2, x, perm)


def _stats_body(a_ref, r_ref, c_ref, abf_ref, *, blk):
    i = pl.program_id(0)
    k = pl.program_id(1)
    a = a_ref[...].astype(jnp.float32)

    @pl.when(i == k)
    def _():
        rr = jax.lax.broadcasted_iota(jnp.int32, (blk, blk), 0)
        cc = jax.lax.broadcasted_iota(jnp.int32, (blk, blk), 1)
        abf_ref[...] = jnp.where(rr == cc, 1.0, a).astype(jnp.bfloat16)

    @pl.when(i != k)
    def _():
        abf_ref[...] = a.astype(jnp.bfloat16)

    @pl.when(k == 0)
    def _():
        r_ref[...] = jnp.zeros_like(r_ref)
        c_ref[...] = jnp.zeros_like(c_ref)

    r_ref[...] += jnp.sum(a, axis=1, keepdims=True) + jnp.zeros(
        (blk, 128), jnp.float32)

    @pl.when(i == k)
    def _():
        rr = jax.lax.broadcasted_iota(jnp.int32, (blk, blk), 0)
        cc = jax.lax.broadcasted_iota(jnp.int32, (blk, blk), 1)
        c_ref[...] += jnp.sum(jnp.where(rr == cc, a, 0.0), axis=1,
                              keepdims=True) + jnp.zeros((blk, 128),
                                                         jnp.float32)


def _stats(a):
    n = a.shape[0]
    blk = min(512, n)
    r, c, abf = pl.pallas_call(
        functools.partial(_stats_body, blk=blk),
        out_shape=[
            jax.ShapeDtypeStruct((n, 128), jnp.float32),
            jax.ShapeDtypeStruct((n, 128), jnp.float32),
            jax.ShapeDtypeStruct((n, n), jnp.bfloat16),
        ],
        grid=(n // blk, n // blk),
        in_specs=[pl.BlockSpec((blk, blk), lambda i, k: (i, k))],
        out_specs=[
            pl.BlockSpec((blk, 128), lambda i, k: (i, 0)),
            pl.BlockSpec((blk, 128), lambda i, k: (i, 0)),
            pl.BlockSpec((blk, blk), lambda i, k: (i, k)),
        ],
    )(a)
    return r[:, 0], c[:, 0], abf


def _norm_vecs(r, c):
    extra = jnp.where(c == 0, 2.0, 0.0)
    deg = r + extra
    dinv = jnp.where(deg > 0, jax.lax.rsqrt(deg), 0.0)
    coeff = (c - 1.0 + extra) * dinv * dinv
    return dinv, coeff


def _gcn_conv(a_raw, dinv, coeff, x, W, b, relu, row_scale=None):
    if row_scale is not None:
        x = x * row_scale[:, None]
    z = _mm(x, W, bn=128)
    zs = z * dinv[:, None]
    y = _mm(a_raw, zs, bn=128) * dinv[:, None] + coeff[:, None] * z + b
    if relu:
        y = jax.nn.relu(y)
    return y


def kernel(x, edge_index, W_down0, b_down0, W_down1, b_down1, W_down2,
           b_down2, W_down3, b_down3, p_pool1, p_pool2, p_pool3,
           W_up0, b_up0, W_up1, b_up1, W_up2, b_up2):
    n = x.shape[0]
    A32 = jnp.zeros((n, n), jnp.float32).at[edge_index[1], edge_index[0]].add(
        jnp.ones((edge_index.shape[1],), jnp.float32))

    r, c, A = _stats(A32)
    dinv, coeff = _norm_vecs(r, c)
    x = _gcn_conv(A, dinv, coeff, x, W_down0, b_down0, relu=True)

    xs = [x]
    As = [A]
    norms = [(dinv, coeff)]
    perms = []
    Wd = [(W_down1, b_down1), (W_down2, b_down2), (W_down3, b_down3)]
    ps = [p_pool1, p_pool2, p_pool3]

    for i in range(DEPTH):
        A2 = _augment(A)
        p = ps[i]
        pn = p / jnp.linalg.norm(p)
        P = jnp.zeros((128, 128), jnp.float32).at[:, 0].set(pn)
        score = _mm(x, P, bn=128)[:, 0]
        k = int(math.ceil(RATIO * x.shape[0]))
        vals, perm = jax.lax.top_k(score, k)
        scale = jnp.tanh(vals)
        A, xg, r = _pool_sc(A2, x, perm)
        dinv, coeff = _norm_vecs(r, jnp.zeros_like(r))
        x = _gcn_conv(A, dinv, coeff, xg, Wd[i][0], Wd[i][1], relu=True,
                      row_scale=scale)
        if i < DEPTH - 1:
            xs.append(x)
            As.append(A)
            norms.append((dinv, coeff))
        perms.append(perm)

    Wu = [(W_up0, b_up0), (W_up1, b_up1), (W_up2, b_up2)]
    for i in range(DEPTH):
        j = DEPTH - 1 - i
        res = xs[j]
        A = As[j]
        dinv, coeff = norms[j]
        perm = perms[j]
        Wt, bt = Wu[i]
        h = _mm(res, Wt[:128], bn=128) + jnp.zeros(
            (res.shape[0], Wt.shape[1]), jnp.float32).at[perm].set(
                _mm(x, Wt[128:], bn=128))
        hs = h * dinv[:, None]
        y = _mm(A, hs, bn=128) * dinv[:, None] + coeff[:, None] * h + bt
        if i < DEPTH - 1:
            y = jax.nn.relu(y)
        x = y
    return x
